# baseline (speedup 1.0000x reference)
import jax
import jax.numpy as jnp
from jax.experimental import pallas as pl
from jax.experimental.pallas import tpu as pltpu

_BM = 200


def _gcn_kernel(x_ref, w_ref, a0_ref, a1_ref, o0_ref, o1_ref, support_ref):
    @pl.when(pl.program_id(0) == 0)
    def _compute_support():
        support_ref[...] = jnp.dot(
            x_ref[...], w_ref[...], preferred_element_type=jnp.float32
        )

    o0_ref[...] = jnp.dot(
        a0_ref[...], support_ref[...], preferred_element_type=jnp.float32
    )
    o1_ref[...] = jnp.dot(
        a1_ref[...], support_ref[...], preferred_element_type=jnp.float32
    )


def kernel(inputs, adj, weight):
    n, d_in = inputs.shape
    d_out = weight.shape[1]
    half_blocks = n // (2 * _BM)
    o0, o1 = pl.pallas_call(
        _gcn_kernel,
        grid=(half_blocks,),
        in_specs=[
            pl.BlockSpec((n, d_in), lambda i: (0, 0)),
            pl.BlockSpec((d_in, d_out), lambda i: (0, 0)),
            pl.BlockSpec((_BM, n), lambda i: (i, 0)),
            pl.BlockSpec((_BM, n), lambda i: (i + half_blocks, 0)),
        ],
        out_specs=[
            pl.BlockSpec((_BM, d_out), lambda i: (i, 0)),
            pl.BlockSpec((_BM, d_out), lambda i: (i, 0)),
        ],
        out_shape=[
            jax.ShapeDtypeStruct((n // 2, d_out), jnp.float32),
            jax.ShapeDtypeStruct((n // 2, d_out), jnp.float32),
        ],
        scratch_shapes=[pltpu.VMEM((n, d_out), jnp.float32)],
    )(inputs, weight, adj, adj)
    return jnp.concatenate([o0, o1], axis=0)


# speedup vs baseline: 1.0508x; 1.0508x over previous
import jax
import jax.numpy as jnp
from jax.experimental import pallas as pl
from jax.experimental.pallas import tpu as pltpu

_BM = 400


def _gcn_kernel(x_ref, w_ref, adj_ref, out_ref, support_ref):
    @pl.when(pl.program_id(0) == 0)
    def _compute_support():
        support_ref[...] = jnp.dot(
            x_ref[...], w_ref[...], preferred_element_type=jnp.float32
        )

    out_ref[...] = jnp.dot(
        adj_ref[...], support_ref[...], preferred_element_type=jnp.float32
    )


def kernel(inputs, adj, weight):
    n, d_in = inputs.shape
    d_out = weight.shape[1]
    return pl.pallas_call(
        _gcn_kernel,
        grid=(n // _BM,),
        in_specs=[
            pl.BlockSpec((n, d_in), lambda i: (0, 0)),
            pl.BlockSpec((d_in, d_out), lambda i: (0, 0)),
            pl.BlockSpec((_BM, n), lambda i: (i, 0)),
        ],
        out_specs=pl.BlockSpec((_BM, d_out), lambda i: (i, 0)),
        out_shape=jax.ShapeDtypeStruct((n, d_out), jnp.float32),
        scratch_shapes=[pltpu.VMEM((n, d_out), jnp.float32)],
    )(inputs, weight, adj)
